# baseline (speedup 1.0000x reference)
import functools
import math

import jax
import jax.numpy as jnp
from jax import lax
from jax.experimental import pallas as pl
from jax.experimental.pallas import tpu as pltpu
from jax.experimental.pallas import tpu_sc as plsc

_BR = 256
_L = 16


def _ffn_block_kernel(bmap_ref, meta_ref, xs_ref, w1_ref, b1_ref, w2_ref,
                      b2_ref, rw_ref, out_ref):
    b = pl.program_id(0)

    @pl.when(b < meta_ref[0])
    def _():
        x = xs_ref[...]
        h = jnp.dot(x, w1_ref[0], preferred_element_type=jnp.float32)
        h = h + b1_ref[0]
        h = 0.5 * h * (1.0 + jax.lax.erf(h * (1.0 / math.sqrt(2.0))))
        hb = h.astype(jnp.bfloat16)
        out_ref[...] = (
            jnp.dot(hb, w2_ref[0], preferred_element_type=jnp.float32)
            + b2_ref[0]) * rw_ref[...]


def _grouped_ffn(xs, bmap, meta, w1, b1, w2, b2, rw, nb):
    E, D, H = w1.shape
    P = xs.shape[0]
    grid_spec = pltpu.PrefetchScalarGridSpec(
        num_scalar_prefetch=2,
        grid=(nb,),
        in_specs=[
            pl.BlockSpec((_BR, D), lambda b, bm, mt: (jnp.minimum(b, mt[0] - 1), 0)),
            pl.BlockSpec((1, D, H), lambda b, bm, mt: (bm[b], 0, 0)),
            pl.BlockSpec((1, 1, H), lambda b, bm, mt: (bm[b], 0, 0)),
            pl.BlockSpec((1, H, D), lambda b, bm, mt: (bm[b], 0, 0)),
            pl.BlockSpec((1, 1, D), lambda b, bm, mt: (bm[b], 0, 0)),
            pl.BlockSpec((_BR, 1), lambda b, bm, mt: (jnp.minimum(b, mt[0] - 1), 0)),
        ],
        out_specs=pl.BlockSpec((_BR, D), lambda b, bm, mt: (b, 0)),
    )
    return pl.pallas_call(
        _ffn_block_kernel,
        grid_spec=grid_spec,
        out_shape=jax.ShapeDtypeStruct((P, D), jnp.float32),
        compiler_params=pltpu.CompilerParams(
            dimension_semantics=("arbitrary",),
        ),
    )(bmap, meta, xs, w1, b1.reshape(E, 1, H), w2, b2.reshape(E, 1, D), rw)


def _sc_mesh():
    return plsc.VectorSubcoreMesh(core_axis_name="c", subcore_axis_name="s")


def _wid(info):
    return lax.axis_index("s") * info.num_cores + lax.axis_index("c")


def _make_sc_gather(P, W):
    info = plsc.get_sparse_core_info()
    nw = info.num_cores * info.num_subcores
    rows_per = P // nw
    chunk = rows_per
    while chunk > 128:
        chunk //= 2
    nch = rows_per // chunk

    @functools.partial(
        pl.kernel,
        mesh=_sc_mesh(),
        out_type=jax.ShapeDtypeStruct((P, W), jnp.int32),
        scratch_types=[
            pltpu.VMEM((rows_per,), jnp.int32),
            pltpu.VMEM((rows_per, W), jnp.int32),
            pltpu.SemaphoreType.DMA,
        ],
    )
    def k(x_hbm, tok_hbm, out_hbm, idx_v, rows_v, sem):
        base = _wid(info) * rows_per
        pltpu.sync_copy(tok_hbm.at[pl.ds(base, rows_per)], idx_v)
        cps = [
            pltpu.async_copy(
                x_hbm.at[idx_v.at[pl.ds(c * chunk, chunk)]],
                rows_v.at[pl.ds(c * chunk, chunk)], sem)
            for c in range(nch)
        ]
        for cp in cps:
            cp.wait()
        pltpu.sync_copy(rows_v, out_hbm.at[pl.ds(base, rows_per)])

    return k


def _make_sc_combine(P, T, D):
    info = plsc.get_sparse_core_info()
    nw = info.num_cores * info.num_subcores
    tw = T // nw
    nch = D // _L

    @functools.partial(
        pl.kernel,
        mesh=_sc_mesh(),
        out_type=jax.ShapeDtypeStruct((T, D), jnp.float32),
        scratch_types=[
            pltpu.VMEM((tw,), jnp.int32),
            pltpu.VMEM((tw,), jnp.int32),
            pltpu.VMEM((tw, D), jnp.float32),
            pltpu.VMEM((tw, D), jnp.float32),
            pltpu.SemaphoreType.DMA,
            pltpu.SemaphoreType.DMA,
        ],
    )
    def k(contrib_hbm, pos0_hbm, pos1_hbm, out_hbm, i0_v, i1_v, r0_v, r1_v,
          s0, s1):
        base = _wid(info) * tw
        pltpu.sync_copy(pos0_hbm.at[pl.ds(base, tw)], i0_v)
        pltpu.sync_copy(pos1_hbm.at[pl.ds(base, tw)], i1_v)
        c0 = pltpu.async_copy(contrib_hbm.at[i0_v], r0_v, s0)
        c1 = pltpu.async_copy(contrib_hbm.at[i1_v], r1_v, s1)
        c0.wait()
        c1.wait()

        def trow(t, _):
            def tchunk(c, _):
                sl = pl.ds(c * _L, _L)
                r0_v[t, sl] = r0_v[t, sl] + r1_v[t, sl]
                return 0
            return lax.fori_loop(0, nch, tchunk, 0, unroll=4)

        lax.fori_loop(0, tw, trow, 0)
        pltpu.sync_copy(r0_v, out_hbm.at[pl.ds(base, tw)])

    return k


def kernel(x, gate_w, w1, b1, w2, b2):
    B, S, D = x.shape
    T = B * S
    E, _, H = w1.shape
    x_flat = x.reshape(T, D)

    logits = x_flat @ gate_w
    i1 = jnp.argmax(logits, axis=-1)
    v1 = jnp.max(logits, axis=-1)
    masked = jnp.where(jax.nn.one_hot(i1, E, dtype=bool), -jnp.inf, logits)
    i2 = jnp.argmax(masked, axis=-1)
    v2 = jnp.max(masked, axis=-1)
    e2 = jnp.exp(v2 - v1)
    wt1 = 1.0 / (1.0 + e2)
    wt2 = e2 / (1.0 + e2)

    e_pairs = jnp.stack([i1, i2], axis=1).reshape(-1).astype(jnp.int32)
    onehot = (e_pairs[:, None] == jnp.arange(E, dtype=jnp.int32)[None, :])
    rank = jnp.take_along_axis(
        jnp.cumsum(onehot.astype(jnp.int32), axis=0) - 1,
        e_pairs[:, None], axis=1)[:, 0]
    counts = jnp.sum(onehot, axis=0, dtype=jnp.int32)
    padded = ((counts + _BR - 1) // _BR) * _BR
    pad_cum = jnp.cumsum(padded)
    start = pad_cum - padded
    slot = start[e_pairs] + rank

    nb = (2 * T) // _BR + E
    P = nb * _BR
    nused = pad_cum[-1] // _BR
    row_token = jnp.zeros((P,), jnp.int32).at[slot].set(
        jnp.arange(2 * T, dtype=jnp.int32) // 2)
    row_wt = jnp.zeros((P,), jnp.float32).at[slot].set(
        jnp.stack([wt1, wt2], axis=1).reshape(-1))
    bstart = jnp.arange(nb, dtype=jnp.int32) * _BR
    braw = jnp.minimum(
        jnp.searchsorted(pad_cum, bstart, side="right"), E - 1
    ).astype(jnp.int32)
    bmap = jnp.where(jnp.arange(nb) < nused, braw, braw[nused - 1])
    meta = jnp.array([0], jnp.int32).at[0].set(nused)
    pos = slot.reshape(T, 2)

    x_i32 = jax.lax.bitcast_convert_type(
        x_flat.astype(jnp.bfloat16).reshape(T, D // 2, 2), jnp.int32)
    xs_i32 = _make_sc_gather(P, D // 2)(x_i32, row_token)
    xs = jax.lax.bitcast_convert_type(xs_i32, jnp.bfloat16).reshape(P, D)

    contrib = _grouped_ffn(xs, bmap, meta, w1.astype(jnp.bfloat16), b1,
                           w2.astype(jnp.bfloat16), b2,
                           row_wt.reshape(P, 1), nb)

    out = _make_sc_combine(P, T, D)(
        contrib, pos[:, 0].astype(jnp.int32), pos[:, 1].astype(jnp.int32))
    return out.reshape(B, S, D)


# speedup vs baseline: 2.1096x; 2.1096x over previous
import functools
import math

import jax
import jax.numpy as jnp
from jax.experimental import pallas as pl
from jax.experimental.pallas import tpu as pltpu

_BR = 256


def _ffn_block_kernel(bmap_ref, meta_ref, xs_ref, w1_ref, b1_ref, w2_ref,
                      b2_ref, out_ref, w1b_ref, w2b_ref):
    b = pl.program_id(0)

    @pl.when(b < meta_ref[0])
    def _():
        changed = (b == 0) | (bmap_ref[b] != bmap_ref[jnp.maximum(b - 1, 0)])

        @pl.when(changed)
        def _():
            w1b_ref[...] = w1_ref[0].astype(jnp.bfloat16)
            w2b_ref[...] = w2_ref[0].astype(jnp.bfloat16)

        x = xs_ref[...]
        h = jnp.dot(x, w1b_ref[...], preferred_element_type=jnp.float32)
        h = h + b1_ref[0]
        h = 0.5 * h * (1.0 + jax.lax.erf(h * (1.0 / math.sqrt(2.0))))
        hb = h.astype(jnp.bfloat16)
        out_ref[...] = (
            jnp.dot(hb, w2b_ref[...], preferred_element_type=jnp.float32)
            + b2_ref[0])


def _grouped_ffn(xs, bmap, meta, w1, b1, w2, b2, nb):
    E, D, H = w1.shape
    P = xs.shape[0]
    grid_spec = pltpu.PrefetchScalarGridSpec(
        num_scalar_prefetch=2,
        grid=(nb,),
        in_specs=[
            pl.BlockSpec((_BR, D), lambda b, bm, mt: (jnp.minimum(b, mt[0] - 1), 0)),
            pl.BlockSpec((1, D, H), lambda b, bm, mt: (bm[b], 0, 0)),
            pl.BlockSpec((1, 1, H), lambda b, bm, mt: (bm[b], 0, 0)),
            pl.BlockSpec((1, H, D), lambda b, bm, mt: (bm[b], 0, 0)),
            pl.BlockSpec((1, 1, D), lambda b, bm, mt: (bm[b], 0, 0)),
        ],
        out_specs=pl.BlockSpec((_BR, D), lambda b, bm, mt: (b, 0)),
        scratch_shapes=[
            pltpu.VMEM((D, H), jnp.bfloat16),
            pltpu.VMEM((H, D), jnp.bfloat16),
        ],
    )
    return pl.pallas_call(
        _ffn_block_kernel,
        grid_spec=grid_spec,
        out_shape=jax.ShapeDtypeStruct((P, D), jnp.float32),
        compiler_params=pltpu.CompilerParams(
            dimension_semantics=("arbitrary",),
        ),
    )(bmap, meta, xs, w1, b1.reshape(E, 1, H), w2, b2.reshape(E, 1, D))


def kernel(x, gate_w, w1, b1, w2, b2):
    B, S, D = x.shape
    T = B * S
    E, _, H = w1.shape
    x_flat = x.reshape(T, D)

    logits = x_flat @ gate_w
    i1 = jnp.argmax(logits, axis=-1)
    v1 = jnp.max(logits, axis=-1)
    masked = jnp.where(jax.nn.one_hot(i1, E, dtype=bool), -jnp.inf, logits)
    i2 = jnp.argmax(masked, axis=-1)
    v2 = jnp.max(masked, axis=-1)
    e2 = jnp.exp(v2 - v1)
    wt1 = 1.0 / (1.0 + e2)
    wt2 = e2 / (1.0 + e2)

    e_pairs = jnp.stack([i1, i2], axis=1).reshape(-1).astype(jnp.int32)
    onehot = (e_pairs[:, None] == jnp.arange(E, dtype=jnp.int32)[None, :])
    rank = jnp.take_along_axis(
        jnp.cumsum(onehot.astype(jnp.int32), axis=0) - 1,
        e_pairs[:, None], axis=1)[:, 0]
    counts = jnp.sum(onehot, axis=0, dtype=jnp.int32)
    padded = ((counts + _BR - 1) // _BR) * _BR
    pad_cum = jnp.cumsum(padded)
    start = pad_cum - padded
    slot = start[e_pairs] + rank

    nb = (2 * T) // _BR + E
    P = nb * _BR
    nused = pad_cum[-1] // _BR
    row_token = jnp.zeros((P,), jnp.int32).at[slot].set(
        jnp.arange(2 * T, dtype=jnp.int32) // 2)
    bstart = jnp.arange(nb, dtype=jnp.int32) * _BR
    braw = jnp.minimum(
        jnp.searchsorted(pad_cum, bstart, side="right"), E - 1
    ).astype(jnp.int32)
    bmap = jnp.where(jnp.arange(nb) < nused, braw, braw[nused - 1])
    meta = jnp.array([0], jnp.int32).at[0].set(nused)
    pos = slot.reshape(T, 2)

    xs = x_flat.astype(jnp.bfloat16)[row_token]
    contrib = _grouped_ffn(xs, bmap, meta, w1, b1, w2, b2, nb)
    out = wt1[:, None] * contrib[pos[:, 0]] + wt2[:, None] * contrib[pos[:, 1]]
    return out.reshape(B, S, D)
